# baseline (speedup 1.0000x reference)
import functools

import jax
import jax.numpy as jnp
from jax import lax
from jax.experimental import pallas as pl
from jax.experimental.pallas import tpu as pltpu
from jax.experimental.pallas import tpu_sc as plsc

HIDDEN = 1024
FFN = 2048
NE = 8
T = 2048
TOPK = 2
TM = 128
F_TILE = 512
NF = FFN // F_TILE
NPAD = T * TOPK + NE * TM
NT = NPAD // TM
NW = 32
TPW = T // NW
CH = 256


def _router_body(x_ref, gw_ref, logits_ref, e0_ref, e1_ref, w0_ref, w1_ref,
                 r0_ref, r1_ref, cnt_ref, h_ref):
    x = x_ref[...]
    logits = lax.dot_general(x, gw_ref[...], (((1,), (1,)), ((), ())),
                             preferred_element_type=jnp.float32)
    logits_ref[...] = logits
    col = lax.broadcasted_iota(jnp.int32, logits.shape, 1)
    m1 = jnp.max(logits, axis=1, keepdims=True)
    e0 = jnp.min(jnp.where(logits == m1, col, NE), axis=1, keepdims=True)
    masked = jnp.where(col == e0, jnp.float32(-1e30), logits)
    m2 = jnp.max(masked, axis=1, keepdims=True)
    e1 = jnp.min(jnp.where(masked == m2, col, NE), axis=1, keepdims=True)
    s = jnp.exp(m2 - m1)
    denom = 1.0 + s
    e0_ref[...] = e0
    e1_ref[...] = e1
    w0_ref[...] = 1.0 / denom
    w1_ref[...] = s / denom
    h_ref[...] = ((col == e0) | (col == e1)).astype(jnp.float32)

    ri = lax.broadcasted_iota(jnp.int32, (CH, CH), 0)
    ci = lax.broadcasted_iota(jnp.int32, (CH, CH), 1)
    tri = (ri > ci).astype(jnp.bfloat16)

    def chunk(i, carry):
        sl = pl.ds(i * CH, CH)
        hc = h_ref[sl, :]
        cc = lax.dot_general(tri, hc.astype(jnp.bfloat16),
                             (((1,), (0,)), ((), ())),
                             preferred_element_type=jnp.float32) + carry
        colc = lax.broadcasted_iota(jnp.int32, (CH, NE), 1)
        e0c = e0_ref[sl, :]
        e1c = e1_ref[sl, :]
        zero = jnp.float32(0.0)
        r0_ref[sl, :] = jnp.sum(jnp.where(colc == e0c, cc, zero), axis=1,
                                keepdims=True).astype(jnp.int32)
        r1_ref[sl, :] = jnp.sum(jnp.where(colc == e1c, cc, zero), axis=1,
                                keepdims=True).astype(jnp.int32)
        return carry + jnp.sum(hc, axis=0, keepdims=True)

    cnt_ref[...] = lax.fori_loop(0, T // CH, chunk,
                                 jnp.zeros((1, NE), jnp.float32))


def _grouped_body(te_ref, xs_ref, w1_ref, w3_ref, w2_ref, out_ref,
                  w1b_ref, w3b_ref, w2b_ref):
    f = pl.program_id(0)
    i = pl.program_id(1)
    prev = te_ref[jnp.maximum(i - 1, 0)]
    changed = (i == 0) | (te_ref[i] != prev)

    @pl.when(changed)
    def _cast():
        w1b_ref[...] = w1_ref[0].astype(jnp.bfloat16)
        w3b_ref[...] = w3_ref[0].astype(jnp.bfloat16)
        w2b_ref[...] = w2_ref[0].astype(jnp.bfloat16)

    xb = xs_ref[...].astype(jnp.bfloat16)
    y1 = lax.dot_general(xb, w1b_ref[...], (((1,), (1,)), ((), ())),
                         preferred_element_type=jnp.float32)
    y3 = lax.dot_general(xb, w3b_ref[...], (((1,), (1,)), ((), ())),
                         preferred_element_type=jnp.float32)
    h = ((y1 * lax.logistic(y1)) * y3).astype(jnp.bfloat16)
    yp = lax.dot_general(h, w2b_ref[...], (((1,), (1,)), ((), ())),
                         preferred_element_type=jnp.float32)
    sl = pl.ds(i * TM, TM)

    @pl.when(f == 0)
    def _set():
        out_ref[sl, :] = yp

    @pl.when(f != 0)
    def _acc():
        out_ref[sl, :] += yp


def _combine_body(a_ref, b_ref, w0_ref, w1_ref, o_ref):
    o_ref[...] = a_ref[...] * w0_ref[...] + b_ref[...] * w1_ref[...]


def _sc_mesh():
    return plsc.VectorSubcoreMesh(core_axis_name="c", subcore_axis_name="s")


def _dispatch_scatter(x_f32, pos0, pos1):
    @functools.partial(
        pl.kernel, mesh=_sc_mesh(),
        out_type=jax.ShapeDtypeStruct((NPAD, HIDDEN), jnp.float32),
        scratch_types=[
            pltpu.VMEM((TPW,), jnp.int32),
            pltpu.VMEM((TPW,), jnp.int32),
            pltpu.VMEM((TPW, HIDDEN), jnp.float32),
            pltpu.SemaphoreType.DMA,
        ],
    )
    def k(x_hbm, p0_hbm, p1_hbm, xs_hbm, i0_v, i1_v, rows_v, sem):
        wid = lax.axis_index("s") * 2 + lax.axis_index("c")
        base = wid * TPW
        pltpu.sync_copy(p0_hbm.at[pl.ds(base, TPW)], i0_v)
        pltpu.sync_copy(p1_hbm.at[pl.ds(base, TPW)], i1_v)
        pltpu.sync_copy(x_hbm.at[pl.ds(base, TPW)], rows_v)
        pltpu.async_copy(rows_v, xs_hbm.at[i0_v], sem).wait()
        pltpu.async_copy(rows_v, xs_hbm.at[i1_v], sem).wait()

    return k(x_f32, pos0, pos1)


def _combine_gather(ys, pos0, pos1):
    @functools.partial(
        pl.kernel, mesh=_sc_mesh(),
        out_type=(jax.ShapeDtypeStruct((T, HIDDEN), jnp.float32),
                  jax.ShapeDtypeStruct((T, HIDDEN), jnp.float32)),
        scratch_types=[
            pltpu.VMEM((TPW,), jnp.int32),
            pltpu.VMEM((TPW,), jnp.int32),
            pltpu.VMEM((TPW, HIDDEN), jnp.float32),
            pltpu.SemaphoreType.DMA,
        ],
    )
    def k(ys_hbm, p0_hbm, p1_hbm, a_hbm, b_hbm, i0_v, i1_v, rows_v, sem):
        wid = lax.axis_index("s") * 2 + lax.axis_index("c")
        base = wid * TPW
        pltpu.sync_copy(p0_hbm.at[pl.ds(base, TPW)], i0_v)
        pltpu.sync_copy(p1_hbm.at[pl.ds(base, TPW)], i1_v)
        pltpu.async_copy(ys_hbm.at[i0_v], rows_v, sem).wait()
        pltpu.sync_copy(rows_v, a_hbm.at[pl.ds(base, TPW)])
        pltpu.async_copy(ys_hbm.at[i1_v], rows_v, sem).wait()
        pltpu.sync_copy(rows_v, b_hbm.at[pl.ds(base, TPW)])

    return k(ys, pos0, pos1)


def kernel(hidden_states, gate_w, w1, w3, w2):
    b, s, hd = hidden_states.shape
    x2 = hidden_states.reshape(T, HIDDEN)

    (logits, e0, e1, wt0, wt1, r0, r1, cnt) = pl.pallas_call(
        _router_body,
        out_shape=(
            jax.ShapeDtypeStruct((T, NE), jnp.float32),
            jax.ShapeDtypeStruct((T, 1), jnp.int32),
            jax.ShapeDtypeStruct((T, 1), jnp.int32),
            jax.ShapeDtypeStruct((T, 1), jnp.float32),
            jax.ShapeDtypeStruct((T, 1), jnp.float32),
            jax.ShapeDtypeStruct((T, 1), jnp.int32),
            jax.ShapeDtypeStruct((T, 1), jnp.int32),
            jax.ShapeDtypeStruct((1, NE), jnp.float32),
        ),
        scratch_shapes=[pltpu.VMEM((T, NE), jnp.float32)],
    )(x2, gate_w)

    counts = cnt.reshape(NE).astype(jnp.int32)
    padded = ((counts + TM - 1) // TM) * TM
    offs = jnp.concatenate([jnp.zeros((1,), jnp.int32),
                            jnp.cumsum(padded)[:-1]])
    pos0 = (jnp.take(offs, e0.reshape(T)) + r0.reshape(T)).astype(jnp.int32)
    pos1 = (jnp.take(offs, e1.reshape(T)) + r1.reshape(T)).astype(jnp.int32)
    starts = offs // TM
    tile_expert = (jnp.sum(
        (jnp.arange(NT, dtype=jnp.int32)[:, None] >= starts[None, :])
        .astype(jnp.int32), axis=1) - 1).astype(jnp.int32)

    xs = _dispatch_scatter(x2, pos0, pos1)

    ys = pl.pallas_call(
        _grouped_body,
        grid_spec=pltpu.PrefetchScalarGridSpec(
            num_scalar_prefetch=1,
            grid=(NF, NT),
            in_specs=[
                pl.BlockSpec((TM, HIDDEN), lambda f, i, te: (i, 0)),
                pl.BlockSpec((1, F_TILE, HIDDEN), lambda f, i, te: (te[i], f, 0)),
                pl.BlockSpec((1, F_TILE, HIDDEN), lambda f, i, te: (te[i], f, 0)),
                pl.BlockSpec((1, HIDDEN, F_TILE), lambda f, i, te: (te[i], 0, f)),
            ],
            out_specs=pl.BlockSpec((NPAD, HIDDEN), lambda f, i, te: (0, 0)),
            scratch_shapes=[
                pltpu.VMEM((F_TILE, HIDDEN), jnp.bfloat16),
                pltpu.VMEM((F_TILE, HIDDEN), jnp.bfloat16),
                pltpu.VMEM((HIDDEN, F_TILE), jnp.bfloat16),
            ],
        ),
        out_shape=jax.ShapeDtypeStruct((NPAD, HIDDEN), jnp.float32),
    )(tile_expert, xs, w1, w3, w2)

    a, bb = _combine_gather(ys, pos0, pos1)

    out = pl.pallas_call(
        _combine_body,
        out_shape=jax.ShapeDtypeStruct((T, HIDDEN), jnp.float32),
    )(a, bb, wt0, wt1)

    return out.reshape(b, s, hd), logits


# speedup vs baseline: 1.2833x; 1.2833x over previous
import functools

import jax
import jax.numpy as jnp
from jax import lax
from jax.experimental import pallas as pl
from jax.experimental.pallas import tpu as pltpu
from jax.experimental.pallas import tpu_sc as plsc

HIDDEN = 1024
FFN = 2048
NE = 8
T = 2048
TOPK = 2
TM = 256
F_TILE = 512
NF = FFN // F_TILE
NPAD = T * TOPK + NE * TM
NT = NPAD // TM
NW = 32
TPW = T // NW
CH = 256


def _router_body(x_ref, gw_ref, logits_ref, e0_ref, e1_ref, w0_ref, w1_ref,
                 r0_ref, r1_ref, cnt_ref, h_ref):
    x = x_ref[...]
    logits = lax.dot_general(x, gw_ref[...], (((1,), (1,)), ((), ())),
                             preferred_element_type=jnp.float32)
    logits_ref[...] = logits
    col = lax.broadcasted_iota(jnp.int32, logits.shape, 1)
    m1 = jnp.max(logits, axis=1, keepdims=True)
    e0 = jnp.min(jnp.where(logits == m1, col, NE), axis=1, keepdims=True)
    masked = jnp.where(col == e0, jnp.float32(-1e30), logits)
    m2 = jnp.max(masked, axis=1, keepdims=True)
    e1 = jnp.min(jnp.where(masked == m2, col, NE), axis=1, keepdims=True)
    s = jnp.exp(m2 - m1)
    denom = 1.0 + s
    e0_ref[...] = e0
    e1_ref[...] = e1
    w0_ref[...] = 1.0 / denom
    w1_ref[...] = s / denom
    h_ref[...] = ((col == e0) | (col == e1)).astype(jnp.float32)

    ri = lax.broadcasted_iota(jnp.int32, (CH, CH), 0)
    ci = lax.broadcasted_iota(jnp.int32, (CH, CH), 1)
    tri = (ri > ci).astype(jnp.bfloat16)

    def chunk(i, carry):
        sl = pl.ds(i * CH, CH)
        hc = h_ref[sl, :]
        cc = lax.dot_general(tri, hc.astype(jnp.bfloat16),
                             (((1,), (0,)), ((), ())),
                             preferred_element_type=jnp.float32) + carry
        colc = lax.broadcasted_iota(jnp.int32, (CH, NE), 1)
        e0c = e0_ref[sl, :]
        e1c = e1_ref[sl, :]
        zero = jnp.float32(0.0)
        r0_ref[sl, :] = jnp.sum(jnp.where(colc == e0c, cc, zero), axis=1,
                                keepdims=True).astype(jnp.int32)
        r1_ref[sl, :] = jnp.sum(jnp.where(colc == e1c, cc, zero), axis=1,
                                keepdims=True).astype(jnp.int32)
        return carry + jnp.sum(hc, axis=0, keepdims=True)

    cnt_ref[...] = lax.fori_loop(0, T // CH, chunk,
                                 jnp.zeros((1, NE), jnp.float32))


def _grouped_body(te_ref, xs_ref, w1_ref, w3_ref, w2_ref, out_ref,
                  xsb_ref, w1b_ref, w3b_ref, w2b_ref):
    f = pl.program_id(0)
    i = pl.program_id(1)
    sl = pl.ds(i * TM, TM)
    prev = te_ref[jnp.maximum(i - 1, 0)]
    changed = (i == 0) | (te_ref[i] != prev)

    @pl.when(changed)
    def _cast():
        w1b_ref[...] = w1_ref[0].astype(jnp.bfloat16)
        w3b_ref[...] = w3_ref[0].astype(jnp.bfloat16)
        w2b_ref[...] = w2_ref[0].astype(jnp.bfloat16)

    @pl.when(f == 0)
    def _cx():
        xsb_ref[sl, :] = xs_ref[...].astype(jnp.bfloat16)

    xb = xsb_ref[sl, :]
    y1 = lax.dot_general(xb, w1b_ref[...], (((1,), (1,)), ((), ())),
                         preferred_element_type=jnp.float32)
    y3 = lax.dot_general(xb, w3b_ref[...], (((1,), (1,)), ((), ())),
                         preferred_element_type=jnp.float32)
    h = ((y1 * lax.logistic(y1)) * y3).astype(jnp.bfloat16)
    yp = lax.dot_general(h, w2b_ref[...], (((1,), (1,)), ((), ())),
                         preferred_element_type=jnp.float32)

    @pl.when(f == 0)
    def _set():
        out_ref[sl, :] = yp

    @pl.when(f != 0)
    def _acc():
        out_ref[sl, :] += yp


def _combine_body(a_ref, b_ref, w0_ref, w1_ref, o_ref):
    o_ref[...] = a_ref[...] * w0_ref[...] + b_ref[...] * w1_ref[...]


def _sc_mesh():
    return plsc.VectorSubcoreMesh(core_axis_name="c", subcore_axis_name="s")


def _dispatch_scatter(x_f32, pos0, pos1):
    @functools.partial(
        pl.kernel, mesh=_sc_mesh(),
        out_type=jax.ShapeDtypeStruct((NPAD, HIDDEN), jnp.float32),
        scratch_types=[
            pltpu.VMEM((TPW,), jnp.int32),
            pltpu.VMEM((TPW,), jnp.int32),
            pltpu.VMEM((TPW, HIDDEN), jnp.float32),
            pltpu.SemaphoreType.DMA,
        ],
    )
    def k(x_hbm, p0_hbm, p1_hbm, xs_hbm, i0_v, i1_v, rows_v, sem):
        wid = lax.axis_index("s") * 2 + lax.axis_index("c")
        base = wid * TPW
        pltpu.sync_copy(p0_hbm.at[pl.ds(base, TPW)], i0_v)
        pltpu.sync_copy(p1_hbm.at[pl.ds(base, TPW)], i1_v)
        pltpu.sync_copy(x_hbm.at[pl.ds(base, TPW)], rows_v)
        pltpu.async_copy(rows_v, xs_hbm.at[i0_v], sem).wait()
        pltpu.async_copy(rows_v, xs_hbm.at[i1_v], sem).wait()

    return k(x_f32, pos0, pos1)


def _combine_gather(ys, pos0, pos1):
    @functools.partial(
        pl.kernel, mesh=_sc_mesh(),
        out_type=(jax.ShapeDtypeStruct((T, HIDDEN), jnp.float32),
                  jax.ShapeDtypeStruct((T, HIDDEN), jnp.float32)),
        scratch_types=[
            pltpu.VMEM((TPW,), jnp.int32),
            pltpu.VMEM((TPW,), jnp.int32),
            pltpu.VMEM((TPW, HIDDEN), jnp.float32),
            pltpu.SemaphoreType.DMA,
        ],
    )
    def k(ys_hbm, p0_hbm, p1_hbm, a_hbm, b_hbm, i0_v, i1_v, rows_v, sem):
        wid = lax.axis_index("s") * 2 + lax.axis_index("c")
        base = wid * TPW
        pltpu.sync_copy(p0_hbm.at[pl.ds(base, TPW)], i0_v)
        pltpu.sync_copy(p1_hbm.at[pl.ds(base, TPW)], i1_v)
        pltpu.async_copy(ys_hbm.at[i0_v], rows_v, sem).wait()
        pltpu.sync_copy(rows_v, a_hbm.at[pl.ds(base, TPW)])
        pltpu.async_copy(ys_hbm.at[i1_v], rows_v, sem).wait()
        pltpu.sync_copy(rows_v, b_hbm.at[pl.ds(base, TPW)])

    return k(ys, pos0, pos1)


def kernel(hidden_states, gate_w, w1, w3, w2):
    b, s, hd = hidden_states.shape
    x2 = hidden_states.reshape(T, HIDDEN)

    (logits, e0, e1, wt0, wt1, r0, r1, cnt) = pl.pallas_call(
        _router_body,
        out_shape=(
            jax.ShapeDtypeStruct((T, NE), jnp.float32),
            jax.ShapeDtypeStruct((T, 1), jnp.int32),
            jax.ShapeDtypeStruct((T, 1), jnp.int32),
            jax.ShapeDtypeStruct((T, 1), jnp.float32),
            jax.ShapeDtypeStruct((T, 1), jnp.float32),
            jax.ShapeDtypeStruct((T, 1), jnp.int32),
            jax.ShapeDtypeStruct((T, 1), jnp.int32),
            jax.ShapeDtypeStruct((1, NE), jnp.float32),
        ),
        scratch_shapes=[pltpu.VMEM((T, NE), jnp.float32)],
    )(x2, gate_w)

    counts = cnt.reshape(NE).astype(jnp.int32)
    padded = ((counts + TM - 1) // TM) * TM
    offs = jnp.concatenate([jnp.zeros((1,), jnp.int32),
                            jnp.cumsum(padded)[:-1]])
    pos0 = (jnp.take(offs, e0.reshape(T)) + r0.reshape(T)).astype(jnp.int32)
    pos1 = (jnp.take(offs, e1.reshape(T)) + r1.reshape(T)).astype(jnp.int32)
    starts = offs // TM
    tile_expert = (jnp.sum(
        (jnp.arange(NT, dtype=jnp.int32)[:, None] >= starts[None, :])
        .astype(jnp.int32), axis=1) - 1).astype(jnp.int32)

    xs = _dispatch_scatter(x2, pos0, pos1)

    ys = pl.pallas_call(
        _grouped_body,
        grid_spec=pltpu.PrefetchScalarGridSpec(
            num_scalar_prefetch=1,
            grid=(NF, NT),
            in_specs=[
                pl.BlockSpec((TM, HIDDEN), lambda f, i, te: (i, 0)),
                pl.BlockSpec((1, F_TILE, HIDDEN), lambda f, i, te: (te[i], f, 0)),
                pl.BlockSpec((1, F_TILE, HIDDEN), lambda f, i, te: (te[i], f, 0)),
                pl.BlockSpec((1, HIDDEN, F_TILE), lambda f, i, te: (te[i], 0, f)),
            ],
            out_specs=pl.BlockSpec((NPAD, HIDDEN), lambda f, i, te: (0, 0)),
            scratch_shapes=[
                pltpu.VMEM((NPAD, HIDDEN), jnp.bfloat16),
                pltpu.VMEM((F_TILE, HIDDEN), jnp.bfloat16),
                pltpu.VMEM((F_TILE, HIDDEN), jnp.bfloat16),
                pltpu.VMEM((HIDDEN, F_TILE), jnp.bfloat16),
            ],
        ),
        out_shape=jax.ShapeDtypeStruct((NPAD, HIDDEN), jnp.float32),
    )(tile_expert, xs, w1, w3, w2)

    a, bb = _combine_gather(ys, pos0, pos1)

    out = pl.pallas_call(
        _combine_body,
        out_shape=jax.ShapeDtypeStruct((T, HIDDEN), jnp.float32),
    )(a, bb, wt0, wt1)

    return out.reshape(b, s, hd), logits
